# Initial kernel scaffold; baseline (speedup 1.0000x reference)
import jax
import jax.numpy as jnp
from jax.experimental import pallas as pl


def kernel(x, pos_embeddings):
    raise NotImplementedError("write your pallas kernel here")


# speedup vs baseline: 4.1810x; 4.1810x over previous
import jax
import jax.numpy as jnp
from jax.experimental import pallas as pl


def _bcast_kernel(pos_ref, o_ref):
    o_ref[...] = pos_ref[...][None]


def kernel(x, pos_embeddings):
    B, S = x.shape
    D = pos_embeddings.shape[1]
    block_s = 1024
    grid = (S // block_s, B)
    return pl.pallas_call(
        _bcast_kernel,
        grid=grid,
        in_specs=[pl.BlockSpec((block_s, D), lambda i, b: (i, 0))],
        out_specs=pl.BlockSpec((1, block_s, D), lambda i, b: (b, i, 0)),
        out_shape=jax.ShapeDtypeStruct((B, S, D), pos_embeddings.dtype),
    )(pos_embeddings)


# speedup vs baseline: 5.1842x; 1.2399x over previous
import jax
import jax.numpy as jnp
from jax.experimental import pallas as pl


def _bcast_kernel(pos_ref, o_ref):
    o_ref[...] = jnp.broadcast_to(pos_ref[...][None], o_ref.shape)


def kernel(x, pos_embeddings):
    B, S = x.shape
    D = pos_embeddings.shape[1]
    block_s = 1024
    grid = (S // block_s,)
    return pl.pallas_call(
        _bcast_kernel,
        grid=grid,
        in_specs=[pl.BlockSpec((block_s, D), lambda i: (i, 0))],
        out_specs=pl.BlockSpec((B, block_s, D), lambda i: (0, i, 0)),
        out_shape=jax.ShapeDtypeStruct((B, S, D), pos_embeddings.dtype),
    )(pos_embeddings)
